# Initial kernel scaffold; baseline (speedup 1.0000x reference)
import jax
import jax.numpy as jnp
from jax.experimental import pallas as pl


def kernel(queries, matrix, real_indices):
    raise NotImplementedError("write your pallas kernel here")


# speedup vs baseline: 1.0338x; 1.0338x over previous
import functools

import jax
import jax.numpy as jnp
from jax import lax
from jax.experimental import pallas as pl
from jax.experimental.pallas import tpu as pltpu

_N = 100000
_D = 128
_K = 750
_Q = 1024
_TEMP = 0.5

_BQ = 256
_BN = 2048


def _mm_body(q_ref, m_ref, ip_ref, s_ref):
    q = q_ref[...]
    m = m_ref[...]
    ip = lax.dot_general(
        q.astype(jnp.bfloat16), m.astype(jnp.bfloat16),
        (((1,), (1,)), ((), ())),
        preferred_element_type=jnp.float32,
    )
    msq = jnp.sum(m * m, axis=1)
    ip_ref[...] = ip
    s_ref[...] = 2.0 * ip - msq[None, :]


def _scores(queries, matrix):
    grid = (_Q // _BQ, pl.cdiv(_N, _BN))
    return pl.pallas_call(
        _mm_body,
        grid=grid,
        in_specs=[
            pl.BlockSpec((_BQ, _D), lambda i, j: (i, 0)),
            pl.BlockSpec((_BN, _D), lambda i, j: (j, 0)),
        ],
        out_specs=[
            pl.BlockSpec((_BQ, _BN), lambda i, j: (i, j)),
            pl.BlockSpec((_BQ, _BN), lambda i, j: (i, j)),
        ],
        out_shape=[
            jax.ShapeDtypeStruct((_Q, _N), jnp.float32),
            jax.ShapeDtypeStruct((_Q, _N), jnp.float32),
        ],
    )(queries, matrix)


def kernel(queries, matrix, real_indices):
    ip, s = _scores(queries, matrix)
    _, knn_idx = lax.top_k(s, _K)
    indices = jnp.concatenate(
        [knn_idx, real_indices[:, None].astype(knn_idx.dtype)], axis=1)
    ip_sel = jnp.take_along_axis(ip, indices, axis=1)
    probs = jax.nn.softmax(ip_sel / _TEMP, axis=1)
    top_candidates = jnp.take_along_axis(
        indices, jnp.argmax(probs, axis=1)[:, None], axis=1)[:, 0]
    v = jnp.zeros((_Q, _N), dtype=jnp.float32)
    batch_idx = jnp.arange(_Q)[:, None]
    v = v.at[batch_idx, indices].set(probs)
    return (v, top_candidates)
